# baseline (speedup 1.0000x reference)
import functools

import jax
import jax.numpy as jnp
from jax import lax
from jax.experimental import pallas as pl
from jax.experimental.pallas import tpu as pltpu
from jax.experimental.pallas import tpu_sc as plsc

SEQ = 21
L = 16
HEAD = 6


def _sc_renorm_body(table_hbm, out_hbm, row_v, sem):
    D = row_v.shape[0]
    nc = 2
    wid = lax.axis_index("s") * nc + lax.axis_index("c")
    pltpu.async_copy(table_hbm.at[wid], row_v, sem).wait()

    def _sumsq_step(j, acc):
        v = row_v[pl.ds(j * L, L)]
        return acc + v * v

    acc = lax.fori_loop(0, D // L, _sumsq_step, jnp.zeros((L,), jnp.float32))
    s_v = jnp.full((L,), jnp.sum(acc), jnp.float32)
    i_v = plsc.bitcast(s_v, jnp.int32)
    i_v = jnp.full((L,), 0x5F3759DF, jnp.int32) - (i_v >> 1)
    y = plsc.bitcast(i_v, jnp.float32)
    for _ in range(3):
        y = y * (1.5 - 0.5 * s_v * y * y)
    norm_v = s_v * y
    scale = jnp.where(s_v > 1.0, 1.0 / (norm_v + 1e-7), jnp.full((L,), 1.0))

    def _scale_step(j, _):
        row_v[pl.ds(j * L, L)] = row_v[pl.ds(j * L, L)] * scale
        return 0

    lax.fori_loop(0, D // L, _scale_step, 0)
    pltpu.sync_copy(row_v, out_hbm.at[wid, 0])


def _add_head_body(x_ref, t_ref, o_ref):
    row = t_ref[0, 0, :]
    norm = jnp.sqrt(jnp.sum(row * row))
    scale = jnp.where(norm > 1.0, 1.0 / (norm + 1e-7), 1.0)
    o_ref[...] = x_ref[...] + (row * scale)[None, None, :]


def _add_tail_body(buf_ref, x_ref, t_ref, o_ref):
    del buf_ref
    o_ref[...] = x_ref[...] + t_ref[0, 0, :][None, None, :]


def kernel(x, table, num_people):
    del num_people
    N, T, D = x.shape
    V = table.shape[0]
    n = N // SEQ

    sc_renorm = functools.partial(
        pl.kernel,
        out_type=jax.ShapeDtypeStruct((n, 1, D), jnp.float32),
        mesh=plsc.VectorSubcoreMesh(
            core_axis_name="c", subcore_axis_name="s",
            num_cores=2, num_subcores=16,
        ),
        scratch_types=[
            pltpu.VMEM((D,), jnp.float32),
            pltpu.SemaphoreType.DMA,
        ],
        compiler_params=pltpu.CompilerParams(
            needs_layout_passes=False, skip_device_barrier=True,
        ),
    )(_sc_renorm_body)

    buf = pl.pallas_call(
        _add_head_body,
        grid=(HEAD,),
        in_specs=[
            pl.BlockSpec((SEQ, T, D), lambda i: (i, 0, 0)),
            pl.BlockSpec((1, 1, D), lambda i: (i, 0, 0)),
        ],
        out_specs=pl.BlockSpec((SEQ, T, D), lambda i: (i, 0, 0)),
        out_shape=jax.ShapeDtypeStruct((N, T, D), x.dtype),
    )(x, table.reshape(V, 1, D))

    rows = sc_renorm(table)

    out = pl.pallas_call(
        _add_tail_body,
        grid=(n - HEAD,),
        in_specs=[
            pl.BlockSpec(memory_space=pl.ANY),
            pl.BlockSpec((SEQ, T, D), lambda i: (i + HEAD, 0, 0)),
            pl.BlockSpec((1, 1, D), lambda i: (i + HEAD, 0, 0)),
        ],
        out_specs=pl.BlockSpec((SEQ, T, D), lambda i: (i + HEAD, 0, 0)),
        out_shape=jax.ShapeDtypeStruct((N, T, D), x.dtype),
        input_output_aliases={0: 0},
    )(buf, x, rows)
    return out


# speedup vs baseline: 1.0327x; 1.0327x over previous
import functools

import jax
import jax.numpy as jnp
from jax import lax
from jax.experimental import pallas as pl
from jax.experimental.pallas import tpu as pltpu
from jax.experimental.pallas import tpu_sc as plsc

SEQ = 21
L = 16


def _sc_renorm_body(table_hbm, out_hbm, row_v, sem):
    D = row_v.shape[0]
    nc = 2
    wid = lax.axis_index("s") * nc + lax.axis_index("c")
    pltpu.async_copy(table_hbm.at[wid], row_v, sem).wait()

    def _sumsq_step(j, acc):
        v = row_v[pl.ds(j * L, L)]
        return acc + v * v

    acc = lax.fori_loop(0, D // L, _sumsq_step, jnp.zeros((L,), jnp.float32))
    s_v = jnp.full((L,), jnp.sum(acc), jnp.float32)
    i_v = plsc.bitcast(s_v, jnp.int32)
    i_v = jnp.full((L,), 0x5F3759DF, jnp.int32) - (i_v >> 1)
    y = plsc.bitcast(i_v, jnp.float32)
    for _ in range(3):
        y = y * (1.5 - 0.5 * s_v * y * y)
    norm_v = s_v * y
    scale = jnp.where(s_v > 1.0, 1.0 / (norm_v + 1e-7), jnp.full((L,), 1.0))

    def _scale_step(j, _):
        row_v[pl.ds(j * L, L)] = row_v[pl.ds(j * L, L)] * scale
        return 0

    lax.fori_loop(0, D // L, _scale_step, 0)
    pltpu.sync_copy(row_v, out_hbm.at[wid, 0])


def _add_body(x_ref, t_ref, o_ref):
    o_ref[:SEQ] = x_ref[:SEQ] + t_ref[0, 0, :][None, None, :]
    o_ref[SEQ:] = x_ref[SEQ:] + t_ref[1, 0, :][None, None, :]


def kernel(x, table, num_people):
    del num_people
    N, T, D = x.shape
    n = N // SEQ

    sc_renorm = functools.partial(
        pl.kernel,
        out_type=jax.ShapeDtypeStruct((n, 1, D), jnp.float32),
        mesh=plsc.VectorSubcoreMesh(
            core_axis_name="c", subcore_axis_name="s",
            num_cores=2, num_subcores=16,
        ),
        scratch_types=[
            pltpu.VMEM((D,), jnp.float32),
            pltpu.SemaphoreType.DMA,
        ],
        compiler_params=pltpu.CompilerParams(needs_layout_passes=False),
    )(_sc_renorm_body)
    rows = sc_renorm(table)

    out = pl.pallas_call(
        _add_body,
        grid=(n // 2,),
        in_specs=[
            pl.BlockSpec((2 * SEQ, T, D), lambda i: (i, 0, 0)),
            pl.BlockSpec((2, 1, D), lambda i: (i, 0, 0)),
        ],
        out_specs=pl.BlockSpec((2 * SEQ, T, D), lambda i: (i, 0, 0)),
        out_shape=jax.ShapeDtypeStruct((N, T, D), x.dtype),
    )(x, rows)
    return out


# speedup vs baseline: 1.0342x; 1.0014x over previous
import functools

import jax
import jax.numpy as jnp
from jax import lax
from jax.experimental import pallas as pl
from jax.experimental.pallas import tpu as pltpu
from jax.experimental.pallas import tpu_sc as plsc

SEQ = 21
L = 16


def _sc_renorm_body(table_hbm, out_hbm, row_v, sem):
    D = row_v.shape[0]
    nc = 2
    wid = lax.axis_index("s") * nc + lax.axis_index("c")
    pltpu.async_copy(table_hbm.at[wid], row_v, sem).wait()

    acc = jnp.zeros((L,), jnp.float32)
    for j in range(D // L):
        v = row_v[pl.ds(j * L, L)]
        acc = acc + v * v
    s_v = jnp.full((L,), jnp.sum(acc), jnp.float32)
    i_v = plsc.bitcast(s_v, jnp.int32)
    i_v = jnp.full((L,), 0x5F3759DF, jnp.int32) - (i_v >> 1)
    y = plsc.bitcast(i_v, jnp.float32)
    for _ in range(3):
        y = y * (1.5 - 0.5 * s_v * y * y)
    norm_v = s_v * y
    scale = jnp.where(s_v > 1.0, 1.0 / (norm_v + 1e-7), jnp.full((L,), 1.0))

    for j in range(D // L):
        row_v[pl.ds(j * L, L)] = row_v[pl.ds(j * L, L)] * scale
    pltpu.sync_copy(row_v, out_hbm.at[wid, 0])


def _add_body(x_ref, t_ref, o_ref):
    for p in range(t_ref.shape[0]):
        o_ref[p * SEQ:(p + 1) * SEQ] = (
            x_ref[p * SEQ:(p + 1) * SEQ] + t_ref[p, 0, :][None, None, :]
        )


def kernel(x, table, num_people):
    del num_people
    N, T, D = x.shape
    n = N // SEQ

    sc_renorm = functools.partial(
        pl.kernel,
        out_type=jax.ShapeDtypeStruct((n, 1, D), jnp.float32),
        mesh=plsc.VectorSubcoreMesh(
            core_axis_name="c", subcore_axis_name="s",
            num_cores=2, num_subcores=16,
        ),
        scratch_types=[
            pltpu.VMEM((D,), jnp.float32),
            pltpu.SemaphoreType.DMA,
        ],
        compiler_params=pltpu.CompilerParams(needs_layout_passes=False),
    )(_sc_renorm_body)
    rows = sc_renorm(table)

    pb = 2
    out = pl.pallas_call(
        _add_body,
        grid=(n // pb,),
        in_specs=[
            pl.BlockSpec((pb * SEQ, T, D), lambda i: (i, 0, 0)),
            pl.BlockSpec((pb, 1, D), lambda i: (i, 0, 0)),
        ],
        out_specs=pl.BlockSpec((pb * SEQ, T, D), lambda i: (i, 0, 0)),
        out_shape=jax.ShapeDtypeStruct((N, T, D), x.dtype),
    )(x, rows)
    return out


# speedup vs baseline: 1.0342x; 1.0001x over previous
import functools

import jax
import jax.numpy as jnp
from jax import lax
from jax.experimental import pallas as pl
from jax.experimental.pallas import tpu as pltpu
from jax.experimental.pallas import tpu_sc as plsc

SEQ = 21
L = 16


def _sc_renorm_body(table_hbm, out_hbm, row_v, sem):
    D = row_v.shape[0]
    n_rows = out_hbm.shape[0]
    n_workers = 16
    sid = lax.axis_index("s")
    for r in range(n_rows // n_workers):
        wid = sid * (n_rows // n_workers) + r
        pltpu.async_copy(table_hbm.at[wid], row_v, sem).wait()

        acc = jnp.zeros((L,), jnp.float32)
        for j in range(D // L):
            v = row_v[pl.ds(j * L, L)]
            acc = acc + v * v
        s_v = jnp.full((L,), jnp.sum(acc), jnp.float32)
        i_v = plsc.bitcast(s_v, jnp.int32)
        i_v = jnp.full((L,), 0x5F3759DF, jnp.int32) - (i_v >> 1)
        y = plsc.bitcast(i_v, jnp.float32)
        for _ in range(3):
            y = y * (1.5 - 0.5 * s_v * y * y)
        norm_v = s_v * y
        scale = jnp.where(s_v > 1.0, 1.0 / (norm_v + 1e-7), jnp.full((L,), 1.0))

        for j in range(D // L):
            row_v[pl.ds(j * L, L)] = row_v[pl.ds(j * L, L)] * scale
        pltpu.sync_copy(row_v, out_hbm.at[wid, 0])


def _add_body(x_ref, t_ref, o_ref):
    for p in range(t_ref.shape[0]):
        o_ref[p * SEQ:(p + 1) * SEQ] = (
            x_ref[p * SEQ:(p + 1) * SEQ] + t_ref[p, 0, :][None, None, :]
        )


def kernel(x, table, num_people):
    del num_people
    N, T, D = x.shape
    n = N // SEQ

    sc_renorm = functools.partial(
        pl.kernel,
        out_type=jax.ShapeDtypeStruct((n, 1, D), jnp.float32),
        mesh=plsc.VectorSubcoreMesh(
            core_axis_name="c", subcore_axis_name="s",
            num_cores=1, num_subcores=16,
        ),
        scratch_types=[
            pltpu.VMEM((D,), jnp.float32),
            pltpu.SemaphoreType.DMA,
        ],
        compiler_params=pltpu.CompilerParams(needs_layout_passes=False),
    )(_sc_renorm_body)
    rows = sc_renorm(table)

    pb = 2
    out = pl.pallas_call(
        _add_body,
        grid=(n // pb,),
        in_specs=[
            pl.BlockSpec((pb * SEQ, T, D), lambda i: (i, 0, 0)),
            pl.BlockSpec((pb, 1, D), lambda i: (i, 0, 0)),
        ],
        out_specs=pl.BlockSpec((pb * SEQ, T, D), lambda i: (i, 0, 0)),
        out_shape=jax.ShapeDtypeStruct((N, T, D), x.dtype),
    )(x, rows)
    return out
